# baseline (speedup 1.0000x reference)
import functools

import numpy as np

import jax
import jax.numpy as jnp
from jax import lax
from jax.experimental import pallas as pl
from jax.experimental.pallas import tpu as pltpu
from jax.experimental.pallas import tpu_sc as plsc

N = 10000
E = 320000
F_IN = 128
H = 64
G = 16

NC = 2
NS = 16
NW = NC * NS
CH = 64
NCHT = E // CH
NCHW = NCHT // NW
NTAIL = NCHT - NCHW * NW

ZROW = 624
ZCNT = 640


def _edge_aggregate(tabs, ea, src, dst, zeros, npack):
    W = H * npack
    mesh = plsc.VectorSubcoreMesh(core_axis_name="c", subcore_axis_name="s")
    utabs, slot2tab = [], []
    for t in tabs:
        if not any(t is ut for ut in utabs):
            utabs.append(t)
        slot2tab.append([i for i, ut in enumerate(utabs) if ut is t][0])
    ngather = len(utabs)
    NB = 3 if ngather == 1 else 2

    @functools.partial(
        pl.kernel,
        mesh=mesh,
        compiler_params=pltpu.CompilerParams(use_tc_tiling_on_sc=False,
                                             needs_layout_passes=False),
        out_type=jax.ShapeDtypeStruct((2 * N, W), jnp.float32),
        scratch_types=[
            [pltpu.VMEM((CH,), jnp.int32)] * NB,
            [pltpu.VMEM((CH,), jnp.int32)] * NB,
            [pltpu.VMEM((CH, H), jnp.float32)] * (NB * ngather),
            [pltpu.VMEM((CH, W // 2), jnp.int32)] * NB,
            [pltpu.VMEM((CH, W), jnp.float32)] * NB,
            pltpu.VMEM_SHARED((N, W), jnp.float32),
            [pltpu.SemaphoreType.DMA] * NB,
            [pltpu.SemaphoreType.DMA] * NB,
            [pltpu.SemaphoreType.DMA] * NB,
            [pltpu.SemaphoreType.DMA] * NB,
        ],
    )
    def k(*refs):
        tab_hbms = refs[:ngather]
        ea_hbm, src_hbm, dst_hbm, z_hbm, out_hbm = refs[ngather:ngather + 5]
        (src_vs, dst_vs, xg_vs, eb_vs, m_vs, aggr_sh,
         sem_meta, sem_ea, sem_g, sem_sc) = refs[ngather + 5:]
        cid = lax.axis_index("c")
        sid = lax.axis_index("s")
        wid = cid * NS + sid
        zbase = sid * ZROW

        pltpu.sync_copy(z_hbm.at[pl.ds(zbase, ZCNT)], aggr_sh.at[pl.ds(zbase, ZCNT)])
        plsc.subcore_barrier()

        def issue_eg(kk, b):
            base = (wid * NCHW + kk) * CH
            c1 = pltpu.async_copy(src_hbm.at[pl.ds(base, CH)], src_vs[b], sem_meta[b])
            c2 = pltpu.async_copy(dst_hbm.at[pl.ds(base, CH)], dst_vs[b], sem_meta[b])
            pltpu.async_copy(ea_hbm.at[pl.ds(base, CH)], eb_vs[b], sem_ea[b])
            c1.wait()
            c2.wait()
            for t in range(ngather):
                pltpu.async_copy(tab_hbms[t].at[src_vs[b]], xg_vs[t * NB + b],
                                 sem_g[b])

        def wait_g_ea(b):
            for t in range(ngather):
                pltpu.make_async_copy(tab_hbms[t].at[src_vs[b]],
                                      xg_vs[t * NB + b], sem_g[b]).wait()
            pltpu.make_async_copy(ea_hbm.at[pl.ds(0, CH)], eb_vs[b],
                                  sem_ea[b]).wait()

        def compute(b):
            @plsc.parallel_loop(0, CH, unroll=2)
            def _(r):
                for jh in range(H // 32):
                    g1s = [xg_vs[t * NB + b][r, pl.ds(jh * 32, 16)]
                           for t in range(ngather)]
                    g2s = [xg_vs[t * NB + b][r, pl.ds(jh * 32 + 16, 16)]
                           for t in range(ngather)]
                    for s in range(npack):
                        t = slot2tab[s]
                        v = eb_vs[b][r, pl.ds((s * H + jh * 32) // 2, 16)]
                        a1 = plsc.bitcast(jnp.left_shift(v, 16), jnp.float32)
                        a2 = plsc.bitcast(
                            jnp.bitwise_and(v, jnp.int32(-65536)), jnp.float32)
                        m_vs[b][r, pl.ds(s * H + jh * 32, 16)] = (
                            jnp.maximum(a1 + g1s[t], 0.0))
                        m_vs[b][r, pl.ds(s * H + jh * 32 + 16, 16)] = (
                            jnp.maximum(a2 + g2s[t], 0.0))

        for b in range(NB):
            issue_eg(b, b)

        def pipe_body(m, carry):
            for b in range(NB):
                wait_g_ea(b)
                compute(b)
                pltpu.async_copy(m_vs[b], aggr_sh.at[dst_vs[b]], sem_sc[b],
                                 add=True)
            for b in range(NB):
                pltpu.make_async_copy(m_vs[b], aggr_sh.at[dst_vs[b]],
                                      sem_sc[b]).wait()
                issue_eg(NB * (m + 1) + b, b)
            return carry

        lax.fori_loop(0, NCHW // NB, pipe_body, 0)
        for b in range(NB):
            wait_g_ea(b)

        @pl.when(wid < NTAIL)
        def _():
            base = (NW * NCHW + wid) * CH
            pltpu.sync_copy(src_hbm.at[pl.ds(base, CH)], src_vs[0])
            pltpu.sync_copy(dst_hbm.at[pl.ds(base, CH)], dst_vs[0])
            pltpu.sync_copy(ea_hbm.at[pl.ds(base, CH)], eb_vs[0])
            for t in range(ngather):
                pltpu.async_copy(tab_hbms[t].at[src_vs[0]], xg_vs[t * NB],
                                 sem_g[0]).wait()
            compute(0)
            pltpu.sync_copy(m_vs[0], aggr_sh.at[dst_vs[0]], add=True)

        plsc.subcore_barrier()
        pltpu.sync_copy(aggr_sh.at[pl.ds(zbase, ZCNT)],
                        out_hbm.at[pl.ds(cid * N + zbase, ZCNT)])

    return k(*utabs, ea, src, dst, zeros)


def _edge_transform(ea, w1a, b1a, wps, bps, w1b, b1b):
    BE = 3200
    grid = (E // BE,)
    const2 = lambda i: (0, 0)
    row = lambda i: (i, 0)
    bf16 = jnp.bfloat16
    widths = [w.shape[1] for w in wps]

    def body(ea_ref, w1a_r, b1a_r, w1b_r, b1b_r, *wbo):
        wrs = wbo[0:5]
        brs = wbo[5:10]
        outs = wbo[10:15]
        e = ea_ref[...]
        t1 = jnp.maximum(
            jnp.dot(e, w1a_r[...], preferred_element_type=jnp.float32)
            + b1a_r[0:1, :], 0.0)
        t2 = jnp.maximum(
            jnp.dot(e, w1b_r[...], preferred_element_type=jnp.float32)
            + b1b_r[0:1, :], 0.0)
        for i, t in enumerate([t1, t1, t1, t2, t2]):
            z = jnp.dot(t, wrs[i][...], preferred_element_type=jnp.float32)
            outs[i][...] = (z + brs[i][0:1, :]).astype(bf16)

    in_specs = [
        pl.BlockSpec((BE, 16), row),
        pl.BlockSpec((16, H), const2),
        pl.BlockSpec((8, H), const2),
        pl.BlockSpec((16, H), const2),
        pl.BlockSpec((8, H), const2),
    ]
    in_specs += [pl.BlockSpec((H, w), const2) for w in widths]
    in_specs += [pl.BlockSpec((8, w), const2) for w in widths]
    return pl.pallas_call(
        body,
        grid=grid,
        in_specs=in_specs,
        out_specs=[pl.BlockSpec((BE, w), row) for w in widths],
        out_shape=[jax.ShapeDtypeStruct((E, w), bf16) for w in widths],
    )(ea, w1a, b1a, w1b, b1b, *wps, *bps)


def _node_update(xin, parts, conv_specs, n1w, n1b, n2w, n2b,
                 l1w, l1b, g, beta, Fin):
    BN = 1000
    grid = (N // BN,)
    row = lambda i: (i, 0)
    shift = lambda i: (i + N // BN, 0)
    const2 = lambda i: (0, 0)
    f32 = jnp.float32
    nparts = len(parts)

    def body(*refs):
        x_r = refs[0]
        prefs = refs[1:1 + 2 * nparts]
        (n1wa, n1wb, n1wc, n1ba, n1bb, n1bc,
         n2wa, n2wb, n2wc, n2ba, n2bb, n2bc,
         l1w_r, l1b_r, g_r, beta_r, out_r) = refs[1 + 2 * nparts:]
        x_b = x_r[...]
        psums = [prefs[2 * ph][...] + prefs[2 * ph + 1][...]
                 for ph in range(nparts)]

        def conv(c, w1, b1, w2, b2):
            segs = [psums[pi][:, off:off + wid] for pi, off, wid in conv_specs[c]]
            h = x_b + (jnp.concatenate(segs, axis=1) if len(segs) > 1 else segs[0])
            t = jnp.maximum(
                jnp.dot(h, w1[...], preferred_element_type=f32) + b1[0:1, :], 0.0)
            o = jnp.dot(t, w2[...], preferred_element_type=f32) + b2[0:1, :]
            return jnp.maximum(o, 0.0)

        cat = jnp.concatenate([
            conv(0, n1wa, n1ba, n2wa, n2ba),
            conv(1, n1wb, n1bb, n2wb, n2bb),
            conv(2, n1wc, n1bc, n2wc, n2bc),
        ], axis=1)
        y = jnp.maximum(
            jnp.dot(cat, l1w_r[...], preferred_element_type=f32) + l1b_r[0:1, :], 0.0)
        m = jnp.mean(y, axis=-1, keepdims=True)
        v = jnp.mean((y - m) ** 2, axis=-1, keepdims=True)
        out_r[...] = (y - m) / jnp.sqrt(v + 1e-5) * g_r[0:1, :] + beta_r[0:1, :]

    in_specs = [pl.BlockSpec((BN, Fin), row)]
    operands = [xin]
    for pt in parts:
        wp = pt.shape[1]
        in_specs += [pl.BlockSpec((BN, wp), row), pl.BlockSpec((BN, wp), shift)]
        operands += [pt, pt]
    in_specs += [pl.BlockSpec((Fin, H), const2)] * 3
    in_specs += [pl.BlockSpec((8, H), const2)] * 3
    in_specs += [pl.BlockSpec((H, H), const2)] * 3
    in_specs += [pl.BlockSpec((8, H), const2)] * 3
    in_specs += [
        pl.BlockSpec((3 * H, H), const2),
        pl.BlockSpec((8, H), const2),
        pl.BlockSpec((8, H), const2),
        pl.BlockSpec((8, H), const2),
    ]
    operands += [n1w[0], n1w[1], n1w[2], n1b[0], n1b[1], n1b[2],
                 n2w[0], n2w[1], n2w[2], n2b[0], n2b[1], n2b[2],
                 l1w, l1b, g, beta]
    return pl.pallas_call(
        body,
        grid=grid,
        in_specs=in_specs,
        out_specs=pl.BlockSpec((BN, H), row),
        out_shape=jax.ShapeDtypeStruct((N, H), f32),
    )(*operands)


def _pool_head(x2, batch2d, ones_col, u, fc1w, fc1b, g3, b3, fc2w, fc2b):
    BN = 1000
    grid = (N // BN,)
    row = lambda i: (i, 0)
    const2 = lambda i: (0, 0)
    f32 = jnp.float32

    def body(x_r, b_r, one_r, u_r, w1_r, b1_r, g_r, be_r, w2_r, b2_r, out_r,
             sums, cnts):
        i = pl.program_id(0)

        @pl.when(i == 0)
        def _():
            sums[...] = jnp.zeros_like(sums)
            cnts[...] = jnp.zeros_like(cnts)

        oh = (b_r[...] == lax.broadcasted_iota(jnp.int32, (1, G), 1)).astype(f32)
        sums[...] += lax.dot_general(oh, x_r[...], (((0,), (0,)), ((), ())),
                                     preferred_element_type=f32)
        cnts[...] += lax.dot_general(oh, one_r[...], (((0,), (0,)), ((), ())),
                                     preferred_element_type=f32)

        @pl.when(i == grid[0] - 1)
        def _():
            mean = sums[...] / jnp.maximum(cnts[...], 1.0)
            xf = jnp.concatenate([mean, u_r[...]], axis=1)
            h = jnp.maximum(
                jnp.dot(xf, w1_r[...], preferred_element_type=f32) + b1_r[0:1, :],
                0.0)
            mu = jnp.mean(h, axis=-1, keepdims=True)
            var = jnp.mean((h - mu) ** 2, axis=-1, keepdims=True)
            hn = (h - mu) / jnp.sqrt(var + 1e-5) * g_r[0:1, :] + be_r[0:1, :]
            out_r[...] = jnp.dot(hn, w2_r[...], preferred_element_type=f32) + b2_r[0:1, :]

    return pl.pallas_call(
        body,
        grid=grid,
        in_specs=[
            pl.BlockSpec((BN, H), row),
            pl.BlockSpec((BN, 1), row),
            pl.BlockSpec((BN, 1), row),
            pl.BlockSpec((G, 8), const2),
            pl.BlockSpec((H + 8, 32), const2),
            pl.BlockSpec((8, 32), const2),
            pl.BlockSpec((8, 32), const2),
            pl.BlockSpec((8, 32), const2),
            pl.BlockSpec((32, 1), const2),
            pl.BlockSpec((8, 1), const2),
        ],
        out_specs=pl.BlockSpec((G, 1), const2),
        out_shape=jax.ShapeDtypeStruct((G, 1), f32),
        scratch_shapes=[
            pltpu.VMEM((G, H), f32),
            pltpu.VMEM((G, 1), f32),
        ],
    )(x2, batch2d, ones_col, u, fc1w, fc1b, g3, b3, fc2w, fc2b)


def _b8(b):
    return jnp.broadcast_to(b.reshape(1, -1), (8, b.shape[0]))


def _ileave(cols):
    out = []
    for j in range(0, len(cols), 32):
        blk = cols[j:j + 32]
        for t in range(16):
            out += [blk[t], blk[16 + t]]
    return np.array(out)


def kernel(x, edge_index, edge_attr, u, batch, params):
    p = params
    src = edge_index[0]
    dst = edge_index[1]

    wcat1 = jnp.concatenate([p["conv1a_lin_W"], p["conv1b_lin_W"],
                             p["conv1c_lin_W"]], axis=1)
    bcat1 = jnp.concatenate([p["conv1a_lin_b"], p["conv1b_lin_b"],
                             p["conv1c_lin_b"]], axis=0)
    wc1 = p["emlp1_l2_W"] @ wcat1
    bc1 = p["emlp1_l2_b"] @ wcat1 + bcat1
    wcat2 = jnp.concatenate([p["conv2a_lin_W"], p["conv2b_lin_W"],
                             p["conv2c_lin_W"]], axis=1)
    bcat2 = jnp.concatenate([p["conv2a_lin_b"], p["conv2b_lin_b"],
                             p["conv2c_lin_b"]], axis=0)
    wc2 = p["emlp2_l2_W"] @ wcat2
    bc2 = p["emlp2_l2_b"] @ wcat2 + bcat2

    p1c = _ileave(list(range(0, 64)) + list(range(128, 192)))
    p2c = _ileave(list(range(64, 128)) + list(range(192, 256)))
    p3c = _ileave(list(range(256, 384)))
    abc = _ileave(list(range(0, 128)))
    cc = _ileave(list(range(128, 192)))
    wps = [wc1[:, p1c], wc1[:, p2c], wc1[:, p3c], wc2[:, abc], wc2[:, cc]]
    bps = [_b8(bc1[p1c]), _b8(bc1[p2c]), _b8(bc1[p3c]),
           _b8(bc2[abc]), _b8(bc2[cc])]

    ea_p1, ea_p2, ea_p3, ea_ab, ea_c = [
        lax.bitcast_convert_type(a.reshape(E, a.shape[1] // 2, 2), jnp.int32)
        for a in _edge_transform(
            edge_attr,
            p["emlp1_l1_W"], _b8(p["emlp1_l1_b"]), wps, bps,
            p["emlp2_l1_W"], _b8(p["emlp2_l1_b"]))]

    z128 = jnp.zeros((N, 2 * H), jnp.float32)
    z64 = jnp.zeros((N, H), jnp.float32)
    x_lo = lax.slice(x, (0, 0), (N, H))
    x_hi = lax.slice(x, (0, H), (N, F_IN))

    pp1 = _edge_aggregate([x_lo, x_lo], ea_p1, src, dst, z128, 2)
    pp2 = _edge_aggregate([x_hi, x_hi], ea_p2, src, dst, z128, 2)
    pp3 = _edge_aggregate([x_lo, x_hi], ea_p3, src, dst, z128, 2)
    x1 = _node_update(
        x, [pp1, pp2, pp3],
        [[(0, 0, H), (1, 0, H)],
         [(0, H, H), (1, H, H)],
         [(2, 0, 2 * H)]],
        [p["conv1a_nn1_W"], p["conv1b_nn1_W"], p["conv1c_nn1_W"]],
        [_b8(p["conv1a_nn1_b"]), _b8(p["conv1b_nn1_b"]), _b8(p["conv1c_nn1_b"])],
        [p["conv1a_nn2_W"], p["conv1b_nn2_W"], p["conv1c_nn2_W"]],
        [_b8(p["conv1a_nn2_b"]), _b8(p["conv1b_nn2_b"]), _b8(p["conv1c_nn2_b"])],
        p["lin1_W"], _b8(p["lin1_b"]), _b8(p["ln1_g"]), _b8(p["ln1_beta"]),
        F_IN)

    q_ab = _edge_aggregate([x1, x1], ea_ab, src, dst, z128, 2)
    q_c = _edge_aggregate([x1], ea_c, src, dst, z64, 1)
    x2 = _node_update(
        x1, [q_ab, q_c],
        [[(0, 0, H)], [(0, H, H)], [(1, 0, H)]],
        [p["conv2a_nn1_W"], p["conv2b_nn1_W"], p["conv2c_nn1_W"]],
        [_b8(p["conv2a_nn1_b"]), _b8(p["conv2b_nn1_b"]), _b8(p["conv2c_nn1_b"])],
        [p["conv2a_nn2_W"], p["conv2b_nn2_W"], p["conv2c_nn2_W"]],
        [_b8(p["conv2a_nn2_b"]), _b8(p["conv2b_nn2_b"]), _b8(p["conv2c_nn2_b"])],
        p["lin2_W"], _b8(p["lin2_b"]), _b8(p["ln2_g"]), _b8(p["ln2_beta"]),
        H)

    out = _pool_head(
        x2, batch.reshape(N, 1), jnp.ones((N, 1), jnp.float32), u,
        p["fc1_W"], _b8(p["fc1_b"]), _b8(p["ln3_g"]), _b8(p["ln3_beta"]),
        p["fc2_W"], _b8(p["fc2_b"]))
    return out


# speedup vs baseline: 3.2831x; 3.2831x over previous
import functools

import numpy as np

import jax
import jax.numpy as jnp
from jax import lax
from jax.experimental import pallas as pl
from jax.experimental.pallas import tpu as pltpu
from jax.experimental.pallas import tpu_sc as plsc

N = 10000
E = 320000
F_IN = 128
H = 64
G = 16

NC = 2
NS = 16
NW = NC * NS
CH = 64
NCHT = E // CH
NCHW = NCHT // NW
NTAIL = NCHT - NCHW * NW

ZROW = 624
ZCNT = 640


def _edge_aggregate(tabs, ea, src, dst, zeros, npack):
    W = H * npack
    mesh = plsc.VectorSubcoreMesh(core_axis_name="c", subcore_axis_name="s")
    utabs, slot2tab = [], []
    for t in tabs:
        if not any(t is ut for ut in utabs):
            utabs.append(t)
        slot2tab.append([i for i, ut in enumerate(utabs) if ut is t][0])
    ngather = len(utabs)
    NB = 3 if ngather == 1 else 2

    @functools.partial(
        pl.kernel,
        mesh=mesh,
        compiler_params=pltpu.CompilerParams(use_tc_tiling_on_sc=False),
        out_type=jax.ShapeDtypeStruct((2 * N, W), jnp.float32),
        scratch_types=[
            [pltpu.VMEM((CH,), jnp.int32)] * NB,
            [pltpu.VMEM((CH,), jnp.int32)] * NB,
            [pltpu.VMEM((CH, H), jnp.float32)] * (NB * ngather),
            [pltpu.VMEM((CH, W), jnp.float32)] * NB,
            pltpu.VMEM_SHARED((N, W), jnp.float32),
            [pltpu.SemaphoreType.DMA] * NB,
            [pltpu.SemaphoreType.DMA] * NB,
            [pltpu.SemaphoreType.DMA] * NB,
            [pltpu.SemaphoreType.DMA] * NB,
        ],
    )
    def k(*refs):
        tab_hbms = refs[:ngather]
        ea_hbm, src_hbm, dst_hbm, z_hbm, out_hbm = refs[ngather:ngather + 5]
        (src_vs, dst_vs, xg_vs, m_vs, aggr_sh,
         sem_meta, sem_ea, sem_g, sem_sc) = refs[ngather + 5:]
        cid = lax.axis_index("c")
        sid = lax.axis_index("s")
        wid = cid * NS + sid
        zbase = sid * ZROW

        pltpu.sync_copy(z_hbm.at[pl.ds(zbase, ZCNT)], aggr_sh.at[pl.ds(zbase, ZCNT)])
        plsc.subcore_barrier()

        def issue_eg(kk, b):
            base = (wid * NCHW + kk) * CH
            c1 = pltpu.async_copy(src_hbm.at[pl.ds(base, CH)], src_vs[b], sem_meta[b])
            c2 = pltpu.async_copy(dst_hbm.at[pl.ds(base, CH)], dst_vs[b], sem_meta[b])
            pltpu.async_copy(ea_hbm.at[pl.ds(base, CH)], m_vs[b], sem_ea[b])
            c1.wait()
            c2.wait()
            for t in range(ngather):
                pltpu.async_copy(tab_hbms[t].at[src_vs[b]], xg_vs[t * NB + b],
                                 sem_g[b])

        def wait_g_ea(b):
            for t in range(ngather):
                pltpu.make_async_copy(tab_hbms[t].at[src_vs[b]],
                                      xg_vs[t * NB + b], sem_g[b]).wait()
            pltpu.make_async_copy(ea_hbm.at[pl.ds(0, CH)], m_vs[b],
                                  sem_ea[b]).wait()

        def compute(b):
            @plsc.parallel_loop(0, CH, unroll=2)
            def _(r):
                for j in range(H // 16):
                    gs = [xg_vs[t * NB + b][r, pl.ds(j * 16, 16)]
                          for t in range(ngather)]
                    for s in range(npack):
                        sl = pl.ds(s * H + j * 16, 16)
                        m_vs[b][r, sl] = jnp.maximum(
                            m_vs[b][r, sl] + gs[slot2tab[s]], 0.0)

        for b in range(NB):
            issue_eg(b, b)

        def pipe_body(m, carry):
            for b in range(NB):
                wait_g_ea(b)
                compute(b)
                pltpu.async_copy(m_vs[b], aggr_sh.at[dst_vs[b]], sem_sc[b],
                                 add=True)
            for b in range(NB):
                pltpu.make_async_copy(m_vs[b], aggr_sh.at[dst_vs[b]],
                                      sem_sc[b]).wait()
                issue_eg(NB * (m + 1) + b, b)
            return carry

        lax.fori_loop(0, NCHW // NB, pipe_body, 0)
        for b in range(NB):
            wait_g_ea(b)

        @pl.when(wid < NTAIL)
        def _():
            base = (NW * NCHW + wid) * CH
            pltpu.sync_copy(src_hbm.at[pl.ds(base, CH)], src_vs[0])
            pltpu.sync_copy(dst_hbm.at[pl.ds(base, CH)], dst_vs[0])
            pltpu.sync_copy(ea_hbm.at[pl.ds(base, CH)], m_vs[0])
            for t in range(ngather):
                pltpu.async_copy(tab_hbms[t].at[src_vs[0]], xg_vs[t * NB],
                                 sem_g[0]).wait()
            compute(0)
            pltpu.sync_copy(m_vs[0], aggr_sh.at[dst_vs[0]], add=True)

        plsc.subcore_barrier()
        pltpu.sync_copy(aggr_sh.at[pl.ds(zbase, ZCNT)],
                        out_hbm.at[pl.ds(cid * N + zbase, ZCNT)])

    return k(*utabs, ea, src, dst, zeros)


def _edge_transform(ea, w1a, b1a, wps, bps, w1b, b1b):
    BE = 3200
    grid = (E // BE,)
    const2 = lambda i: (0, 0)
    row = lambda i: (i, 0)
    widths = [w.shape[1] for w in wps]

    def body(ea_ref, w1a_r, b1a_r, w1b_r, b1b_r, *wbo):
        wrs = wbo[0:5]
        brs = wbo[5:10]
        outs = wbo[10:15]
        e = ea_ref[...]
        t1 = jnp.maximum(
            jnp.dot(e, w1a_r[...], preferred_element_type=jnp.float32)
            + b1a_r[0:1, :], 0.0)
        t2 = jnp.maximum(
            jnp.dot(e, w1b_r[...], preferred_element_type=jnp.float32)
            + b1b_r[0:1, :], 0.0)
        for i, t in enumerate([t1, t1, t1, t2, t2]):
            z = jnp.dot(t, wrs[i][...], preferred_element_type=jnp.float32)
            outs[i][...] = z + brs[i][0:1, :]

    in_specs = [
        pl.BlockSpec((BE, 16), row),
        pl.BlockSpec((16, H), const2),
        pl.BlockSpec((8, H), const2),
        pl.BlockSpec((16, H), const2),
        pl.BlockSpec((8, H), const2),
    ]
    in_specs += [pl.BlockSpec((H, w), const2) for w in widths]
    in_specs += [pl.BlockSpec((8, w), const2) for w in widths]
    return pl.pallas_call(
        body,
        grid=grid,
        in_specs=in_specs,
        out_specs=[pl.BlockSpec((BE, w), row) for w in widths],
        out_shape=[jax.ShapeDtypeStruct((E, w), jnp.float32) for w in widths],
    )(ea, w1a, b1a, w1b, b1b, *wps, *bps)


def _node_update(xin, parts, conv_specs, n1w, n1b, n2w, n2b,
                 l1w, l1b, g, beta, Fin):
    BN = 1000
    grid = (N // BN,)
    row = lambda i: (i, 0)
    shift = lambda i: (i + N // BN, 0)
    const2 = lambda i: (0, 0)
    f32 = jnp.float32
    nparts = len(parts)

    def body(*refs):
        x_r = refs[0]
        prefs = refs[1:1 + 2 * nparts]
        (n1wa, n1wb, n1wc, n1ba, n1bb, n1bc,
         n2wa, n2wb, n2wc, n2ba, n2bb, n2bc,
         l1w_r, l1b_r, g_r, beta_r, out_r) = refs[1 + 2 * nparts:]
        x_b = x_r[...]
        psums = [prefs[2 * ph][...] + prefs[2 * ph + 1][...]
                 for ph in range(nparts)]

        def conv(c, w1, b1, w2, b2):
            segs = [psums[pi][:, off:off + wid] for pi, off, wid in conv_specs[c]]
            h = x_b + (jnp.concatenate(segs, axis=1) if len(segs) > 1 else segs[0])
            t = jnp.maximum(
                jnp.dot(h, w1[...], preferred_element_type=f32) + b1[0:1, :], 0.0)
            o = jnp.dot(t, w2[...], preferred_element_type=f32) + b2[0:1, :]
            return jnp.maximum(o, 0.0)

        cat = jnp.concatenate([
            conv(0, n1wa, n1ba, n2wa, n2ba),
            conv(1, n1wb, n1bb, n2wb, n2bb),
            conv(2, n1wc, n1bc, n2wc, n2bc),
        ], axis=1)
        y = jnp.maximum(
            jnp.dot(cat, l1w_r[...], preferred_element_type=f32) + l1b_r[0:1, :], 0.0)
        m = jnp.mean(y, axis=-1, keepdims=True)
        v = jnp.mean((y - m) ** 2, axis=-1, keepdims=True)
        out_r[...] = (y - m) / jnp.sqrt(v + 1e-5) * g_r[0:1, :] + beta_r[0:1, :]

    in_specs = [pl.BlockSpec((BN, Fin), row)]
    operands = [xin]
    for pt in parts:
        wp = pt.shape[1]
        in_specs += [pl.BlockSpec((BN, wp), row), pl.BlockSpec((BN, wp), shift)]
        operands += [pt, pt]
    in_specs += [pl.BlockSpec((Fin, H), const2)] * 3
    in_specs += [pl.BlockSpec((8, H), const2)] * 3
    in_specs += [pl.BlockSpec((H, H), const2)] * 3
    in_specs += [pl.BlockSpec((8, H), const2)] * 3
    in_specs += [
        pl.BlockSpec((3 * H, H), const2),
        pl.BlockSpec((8, H), const2),
        pl.BlockSpec((8, H), const2),
        pl.BlockSpec((8, H), const2),
    ]
    operands += [n1w[0], n1w[1], n1w[2], n1b[0], n1b[1], n1b[2],
                 n2w[0], n2w[1], n2w[2], n2b[0], n2b[1], n2b[2],
                 l1w, l1b, g, beta]
    return pl.pallas_call(
        body,
        grid=grid,
        in_specs=in_specs,
        out_specs=pl.BlockSpec((BN, H), row),
        out_shape=jax.ShapeDtypeStruct((N, H), f32),
    )(*operands)


def _pool_head(x2, batch2d, ones_col, u, fc1w, fc1b, g3, b3, fc2w, fc2b):
    BN = 1000
    grid = (N // BN,)
    row = lambda i: (i, 0)
    const2 = lambda i: (0, 0)
    f32 = jnp.float32

    def body(x_r, b_r, one_r, u_r, w1_r, b1_r, g_r, be_r, w2_r, b2_r, out_r,
             sums, cnts):
        i = pl.program_id(0)

        @pl.when(i == 0)
        def _():
            sums[...] = jnp.zeros_like(sums)
            cnts[...] = jnp.zeros_like(cnts)

        oh = (b_r[...] == lax.broadcasted_iota(jnp.int32, (1, G), 1)).astype(f32)
        sums[...] += lax.dot_general(oh, x_r[...], (((0,), (0,)), ((), ())),
                                     preferred_element_type=f32)
        cnts[...] += lax.dot_general(oh, one_r[...], (((0,), (0,)), ((), ())),
                                     preferred_element_type=f32)

        @pl.when(i == grid[0] - 1)
        def _():
            mean = sums[...] / jnp.maximum(cnts[...], 1.0)
            xf = jnp.concatenate([mean, u_r[...]], axis=1)
            h = jnp.maximum(
                jnp.dot(xf, w1_r[...], preferred_element_type=f32) + b1_r[0:1, :],
                0.0)
            mu = jnp.mean(h, axis=-1, keepdims=True)
            var = jnp.mean((h - mu) ** 2, axis=-1, keepdims=True)
            hn = (h - mu) / jnp.sqrt(var + 1e-5) * g_r[0:1, :] + be_r[0:1, :]
            out_r[...] = jnp.dot(hn, w2_r[...], preferred_element_type=f32) + b2_r[0:1, :]

    return pl.pallas_call(
        body,
        grid=grid,
        in_specs=[
            pl.BlockSpec((BN, H), row),
            pl.BlockSpec((BN, 1), row),
            pl.BlockSpec((BN, 1), row),
            pl.BlockSpec((G, 8), const2),
            pl.BlockSpec((H + 8, 32), const2),
            pl.BlockSpec((8, 32), const2),
            pl.BlockSpec((8, 32), const2),
            pl.BlockSpec((8, 32), const2),
            pl.BlockSpec((32, 1), const2),
            pl.BlockSpec((8, 1), const2),
        ],
        out_specs=pl.BlockSpec((G, 1), const2),
        out_shape=jax.ShapeDtypeStruct((G, 1), f32),
        scratch_shapes=[
            pltpu.VMEM((G, H), f32),
            pltpu.VMEM((G, 1), f32),
        ],
    )(x2, batch2d, ones_col, u, fc1w, fc1b, g3, b3, fc2w, fc2b)


def _b8(b):
    return jnp.broadcast_to(b.reshape(1, -1), (8, b.shape[0]))


def kernel(x, edge_index, edge_attr, u, batch, params):
    p = params
    src = edge_index[0]
    dst = edge_index[1]

    wcat1 = jnp.concatenate([p["conv1a_lin_W"], p["conv1b_lin_W"],
                             p["conv1c_lin_W"]], axis=1)
    bcat1 = jnp.concatenate([p["conv1a_lin_b"], p["conv1b_lin_b"],
                             p["conv1c_lin_b"]], axis=0)
    wc1 = p["emlp1_l2_W"] @ wcat1
    bc1 = p["emlp1_l2_b"] @ wcat1 + bcat1
    wcat2 = jnp.concatenate([p["conv2a_lin_W"], p["conv2b_lin_W"],
                             p["conv2c_lin_W"]], axis=1)
    bcat2 = jnp.concatenate([p["conv2a_lin_b"], p["conv2b_lin_b"],
                             p["conv2c_lin_b"]], axis=0)
    wc2 = p["emlp2_l2_W"] @ wcat2
    bc2 = p["emlp2_l2_b"] @ wcat2 + bcat2

    p1c = np.array(list(range(0, 64)) + list(range(128, 192)))
    p2c = np.array(list(range(64, 128)) + list(range(192, 256)))
    p3c = np.array(list(range(256, 384)))
    abc = np.array(list(range(0, 128)))
    cc = np.array(list(range(128, 192)))
    wps = [wc1[:, p1c], wc1[:, p2c], wc1[:, p3c], wc2[:, abc], wc2[:, cc]]
    bps = [_b8(bc1[p1c]), _b8(bc1[p2c]), _b8(bc1[p3c]),
           _b8(bc2[abc]), _b8(bc2[cc])]

    ea_p1, ea_p2, ea_p3, ea_ab, ea_c = _edge_transform(
        edge_attr,
        p["emlp1_l1_W"], _b8(p["emlp1_l1_b"]), wps, bps,
        p["emlp2_l1_W"], _b8(p["emlp2_l1_b"]))

    z128 = jnp.zeros((N, 2 * H), jnp.float32)
    z64 = jnp.zeros((N, H), jnp.float32)
    x_lo = lax.slice(x, (0, 0), (N, H))
    x_hi = lax.slice(x, (0, H), (N, F_IN))

    pp1 = _edge_aggregate([x_lo, x_lo], ea_p1, src, dst, z128, 2)
    pp2 = _edge_aggregate([x_hi, x_hi], ea_p2, src, dst, z128, 2)
    pp3 = _edge_aggregate([x_lo, x_hi], ea_p3, src, dst, z128, 2)
    x1 = _node_update(
        x, [pp1, pp2, pp3],
        [[(0, 0, H), (1, 0, H)],
         [(0, H, H), (1, H, H)],
         [(2, 0, 2 * H)]],
        [p["conv1a_nn1_W"], p["conv1b_nn1_W"], p["conv1c_nn1_W"]],
        [_b8(p["conv1a_nn1_b"]), _b8(p["conv1b_nn1_b"]), _b8(p["conv1c_nn1_b"])],
        [p["conv1a_nn2_W"], p["conv1b_nn2_W"], p["conv1c_nn2_W"]],
        [_b8(p["conv1a_nn2_b"]), _b8(p["conv1b_nn2_b"]), _b8(p["conv1c_nn2_b"])],
        p["lin1_W"], _b8(p["lin1_b"]), _b8(p["ln1_g"]), _b8(p["ln1_beta"]),
        F_IN)

    q_ab = _edge_aggregate([x1, x1], ea_ab, src, dst, z128, 2)
    q_c = _edge_aggregate([x1], ea_c, src, dst, z64, 1)
    x2 = _node_update(
        x1, [q_ab, q_c],
        [[(0, 0, H)], [(0, H, H)], [(1, 0, H)]],
        [p["conv2a_nn1_W"], p["conv2b_nn1_W"], p["conv2c_nn1_W"]],
        [_b8(p["conv2a_nn1_b"]), _b8(p["conv2b_nn1_b"]), _b8(p["conv2c_nn1_b"])],
        [p["conv2a_nn2_W"], p["conv2b_nn2_W"], p["conv2c_nn2_W"]],
        [_b8(p["conv2a_nn2_b"]), _b8(p["conv2b_nn2_b"]), _b8(p["conv2c_nn2_b"])],
        p["lin2_W"], _b8(p["lin2_b"]), _b8(p["ln2_g"]), _b8(p["ln2_beta"]),
        H)

    out = _pool_head(
        x2, batch.reshape(N, 1), jnp.ones((N, 1), jnp.float32), u,
        p["fc1_W"], _b8(p["fc1_b"]), _b8(p["ln3_g"]), _b8(p["ln3_beta"]),
        p["fc2_W"], _b8(p["fc2_b"]))
    return out


# speedup vs baseline: 3.3482x; 1.0198x over previous
import functools

import numpy as np

import jax
import jax.numpy as jnp
from jax import lax
from jax.experimental import pallas as pl
from jax.experimental.pallas import tpu as pltpu
from jax.experimental.pallas import tpu_sc as plsc

N = 10000
E = 320000
F_IN = 128
H = 64
G = 16

NC = 2
NS = 16
NW = NC * NS

ZROW = 624
ZCNT = 640


def _edge_aggregate(tabs, ea, src, dst, zeros, npack):
    W = H * npack
    mesh = plsc.VectorSubcoreMesh(core_axis_name="c", subcore_axis_name="s")
    utabs, slot2tab = [], []
    for t in tabs:
        if not any(t is ut for ut in utabs):
            utabs.append(t)
        slot2tab.append([i for i, ut in enumerate(utabs) if ut is t][0])
    ngather = len(utabs)
    CH = 128 if ngather == 1 else 64
    NB = 2
    NCHT = E // CH
    NCHW = NCHT // NW
    NTAIL = NCHT - NCHW * NW

    @functools.partial(
        pl.kernel,
        mesh=mesh,
        compiler_params=pltpu.CompilerParams(use_tc_tiling_on_sc=False),
        out_type=jax.ShapeDtypeStruct((2 * N, W), jnp.float32),
        scratch_types=[
            [pltpu.VMEM((CH,), jnp.int32)] * NB,
            [pltpu.VMEM((CH,), jnp.int32)] * NB,
            [pltpu.VMEM((CH, H), jnp.float32)] * (NB * ngather),
            [pltpu.VMEM((CH, W), jnp.float32)] * NB,
            pltpu.VMEM_SHARED((N, W), jnp.float32),
            [pltpu.SemaphoreType.DMA] * NB,
            [pltpu.SemaphoreType.DMA] * NB,
            [pltpu.SemaphoreType.DMA] * NB,
            [pltpu.SemaphoreType.DMA] * NB,
        ],
    )
    def k(*refs):
        tab_hbms = refs[:ngather]
        ea_hbm, src_hbm, dst_hbm, z_hbm, out_hbm = refs[ngather:ngather + 5]
        (src_vs, dst_vs, xg_vs, m_vs, aggr_sh,
         sem_meta, sem_ea, sem_g, sem_sc) = refs[ngather + 5:]
        cid = lax.axis_index("c")
        sid = lax.axis_index("s")
        wid = cid * NS + sid
        zbase = sid * ZROW

        pltpu.sync_copy(z_hbm.at[pl.ds(zbase, ZCNT)], aggr_sh.at[pl.ds(zbase, ZCNT)])
        plsc.subcore_barrier()

        def issue_eg(kk, b):
            base = (wid * NCHW + kk) * CH
            c1 = pltpu.async_copy(src_hbm.at[pl.ds(base, CH)], src_vs[b], sem_meta[b])
            c2 = pltpu.async_copy(dst_hbm.at[pl.ds(base, CH)], dst_vs[b], sem_meta[b])
            pltpu.async_copy(ea_hbm.at[pl.ds(base, CH)], m_vs[b], sem_ea[b])
            c1.wait()
            c2.wait()
            for t in range(ngather):
                pltpu.async_copy(tab_hbms[t].at[src_vs[b]], xg_vs[t * NB + b],
                                 sem_g[b])

        def wait_g_ea(b):
            for t in range(ngather):
                pltpu.make_async_copy(tab_hbms[t].at[src_vs[b]],
                                      xg_vs[t * NB + b], sem_g[b]).wait()
            pltpu.make_async_copy(ea_hbm.at[pl.ds(0, CH)], m_vs[b],
                                  sem_ea[b]).wait()

        def compute(b):
            @plsc.parallel_loop(0, CH, unroll=2)
            def _(r):
                for j in range(H // 16):
                    gs = [xg_vs[t * NB + b][r, pl.ds(j * 16, 16)]
                          for t in range(ngather)]
                    for s in range(npack):
                        sl = pl.ds(s * H + j * 16, 16)
                        m_vs[b][r, sl] = jnp.maximum(
                            m_vs[b][r, sl] + gs[slot2tab[s]], 0.0)

        for b in range(NB):
            issue_eg(b, b)

        def pipe_body(m, carry):
            for b in range(NB):
                wait_g_ea(b)
                compute(b)
                pltpu.async_copy(m_vs[b], aggr_sh.at[dst_vs[b]], sem_sc[b],
                                 add=True)
            for b in range(NB):
                pltpu.make_async_copy(m_vs[b], aggr_sh.at[dst_vs[b]],
                                      sem_sc[b]).wait()
                issue_eg(NB * (m + 1) + b, b)
            return carry

        lax.fori_loop(0, NCHW // NB, pipe_body, 0)
        for b in range(NB):
            wait_g_ea(b)

        @pl.when(wid < NTAIL)
        def _():
            base = (NW * NCHW + wid) * CH
            pltpu.sync_copy(src_hbm.at[pl.ds(base, CH)], src_vs[0])
            pltpu.sync_copy(dst_hbm.at[pl.ds(base, CH)], dst_vs[0])
            pltpu.sync_copy(ea_hbm.at[pl.ds(base, CH)], m_vs[0])
            for t in range(ngather):
                pltpu.async_copy(tab_hbms[t].at[src_vs[0]], xg_vs[t * NB],
                                 sem_g[0]).wait()
            compute(0)
            pltpu.sync_copy(m_vs[0], aggr_sh.at[dst_vs[0]], add=True)

        plsc.subcore_barrier()
        pltpu.sync_copy(aggr_sh.at[pl.ds(zbase, ZCNT)],
                        out_hbm.at[pl.ds(cid * N + zbase, ZCNT)])

    return k(*utabs, ea, src, dst, zeros)


def _edge_transform(ea, w1a, b1a, wps, bps, w1b, b1b):
    BE = 3200
    grid = (E // BE,)
    const2 = lambda i: (0, 0)
    row = lambda i: (i, 0)
    widths = [w.shape[1] for w in wps]

    def body(ea_ref, w1a_r, b1a_r, w1b_r, b1b_r, *wbo):
        wrs = wbo[0:5]
        brs = wbo[5:10]
        outs = wbo[10:15]
        e = ea_ref[...]
        t1 = jnp.maximum(
            jnp.dot(e, w1a_r[...], preferred_element_type=jnp.float32)
            + b1a_r[0:1, :], 0.0)
        t2 = jnp.maximum(
            jnp.dot(e, w1b_r[...], preferred_element_type=jnp.float32)
            + b1b_r[0:1, :], 0.0)
        for i, t in enumerate([t1, t1, t1, t2, t2]):
            z = jnp.dot(t, wrs[i][...], preferred_element_type=jnp.float32)
            outs[i][...] = z + brs[i][0:1, :]

    in_specs = [
        pl.BlockSpec((BE, 16), row),
        pl.BlockSpec((16, H), const2),
        pl.BlockSpec((8, H), const2),
        pl.BlockSpec((16, H), const2),
        pl.BlockSpec((8, H), const2),
    ]
    in_specs += [pl.BlockSpec((H, w), const2) for w in widths]
    in_specs += [pl.BlockSpec((8, w), const2) for w in widths]
    return pl.pallas_call(
        body,
        grid=grid,
        in_specs=in_specs,
        out_specs=[pl.BlockSpec((BE, w), row) for w in widths],
        out_shape=[jax.ShapeDtypeStruct((E, w), jnp.float32) for w in widths],
    )(ea, w1a, b1a, w1b, b1b, *wps, *bps)


def _node_update(xin, parts, conv_specs, n1w, n1b, n2w, n2b,
                 l1w, l1b, g, beta, Fin):
    BN = 1000
    grid = (N // BN,)
    row = lambda i: (i, 0)
    shift = lambda i: (i + N // BN, 0)
    const2 = lambda i: (0, 0)
    f32 = jnp.float32
    nparts = len(parts)

    def body(*refs):
        x_r = refs[0]
        prefs = refs[1:1 + 2 * nparts]
        (n1wa, n1wb, n1wc, n1ba, n1bb, n1bc,
         n2wa, n2wb, n2wc, n2ba, n2bb, n2bc,
         l1w_r, l1b_r, g_r, beta_r, out_r) = refs[1 + 2 * nparts:]
        x_b = x_r[...]
        psums = [prefs[2 * ph][...] + prefs[2 * ph + 1][...]
                 for ph in range(nparts)]

        def conv(c, w1, b1, w2, b2):
            segs = [psums[pi][:, off:off + wid] for pi, off, wid in conv_specs[c]]
            h = x_b + (jnp.concatenate(segs, axis=1) if len(segs) > 1 else segs[0])
            t = jnp.maximum(
                jnp.dot(h, w1[...], preferred_element_type=f32) + b1[0:1, :], 0.0)
            o = jnp.dot(t, w2[...], preferred_element_type=f32) + b2[0:1, :]
            return jnp.maximum(o, 0.0)

        cat = jnp.concatenate([
            conv(0, n1wa, n1ba, n2wa, n2ba),
            conv(1, n1wb, n1bb, n2wb, n2bb),
            conv(2, n1wc, n1bc, n2wc, n2bc),
        ], axis=1)
        y = jnp.maximum(
            jnp.dot(cat, l1w_r[...], preferred_element_type=f32) + l1b_r[0:1, :], 0.0)
        m = jnp.mean(y, axis=-1, keepdims=True)
        v = jnp.mean((y - m) ** 2, axis=-1, keepdims=True)
        out_r[...] = (y - m) / jnp.sqrt(v + 1e-5) * g_r[0:1, :] + beta_r[0:1, :]

    in_specs = [pl.BlockSpec((BN, Fin), row)]
    operands = [xin]
    for pt in parts:
        wp = pt.shape[1]
        in_specs += [pl.BlockSpec((BN, wp), row), pl.BlockSpec((BN, wp), shift)]
        operands += [pt, pt]
    in_specs += [pl.BlockSpec((Fin, H), const2)] * 3
    in_specs += [pl.BlockSpec((8, H), const2)] * 3
    in_specs += [pl.BlockSpec((H, H), const2)] * 3
    in_specs += [pl.BlockSpec((8, H), const2)] * 3
    in_specs += [
        pl.BlockSpec((3 * H, H), const2),
        pl.BlockSpec((8, H), const2),
        pl.BlockSpec((8, H), const2),
        pl.BlockSpec((8, H), const2),
    ]
    operands += [n1w[0], n1w[1], n1w[2], n1b[0], n1b[1], n1b[2],
                 n2w[0], n2w[1], n2w[2], n2b[0], n2b[1], n2b[2],
                 l1w, l1b, g, beta]
    return pl.pallas_call(
        body,
        grid=grid,
        in_specs=in_specs,
        out_specs=pl.BlockSpec((BN, H), row),
        out_shape=jax.ShapeDtypeStruct((N, H), f32),
    )(*operands)


def _pool_head(x2, batch2d, ones_col, u, fc1w, fc1b, g3, b3, fc2w, fc2b):
    BN = 1000
    grid = (N // BN,)
    row = lambda i: (i, 0)
    const2 = lambda i: (0, 0)
    f32 = jnp.float32

    def body(x_r, b_r, one_r, u_r, w1_r, b1_r, g_r, be_r, w2_r, b2_r, out_r,
             sums, cnts):
        i = pl.program_id(0)

        @pl.when(i == 0)
        def _():
            sums[...] = jnp.zeros_like(sums)
            cnts[...] = jnp.zeros_like(cnts)

        oh = (b_r[...] == lax.broadcasted_iota(jnp.int32, (1, G), 1)).astype(f32)
        sums[...] += lax.dot_general(oh, x_r[...], (((0,), (0,)), ((), ())),
                                     preferred_element_type=f32)
        cnts[...] += lax.dot_general(oh, one_r[...], (((0,), (0,)), ((), ())),
                                     preferred_element_type=f32)

        @pl.when(i == grid[0] - 1)
        def _():
            mean = sums[...] / jnp.maximum(cnts[...], 1.0)
            xf = jnp.concatenate([mean, u_r[...]], axis=1)
            h = jnp.maximum(
                jnp.dot(xf, w1_r[...], preferred_element_type=f32) + b1_r[0:1, :],
                0.0)
            mu = jnp.mean(h, axis=-1, keepdims=True)
            var = jnp.mean((h - mu) ** 2, axis=-1, keepdims=True)
            hn = (h - mu) / jnp.sqrt(var + 1e-5) * g_r[0:1, :] + be_r[0:1, :]
            out_r[...] = jnp.dot(hn, w2_r[...], preferred_element_type=f32) + b2_r[0:1, :]

    return pl.pallas_call(
        body,
        grid=grid,
        in_specs=[
            pl.BlockSpec((BN, H), row),
            pl.BlockSpec((BN, 1), row),
            pl.BlockSpec((BN, 1), row),
            pl.BlockSpec((G, 8), const2),
            pl.BlockSpec((H + 8, 32), const2),
            pl.BlockSpec((8, 32), const2),
            pl.BlockSpec((8, 32), const2),
            pl.BlockSpec((8, 32), const2),
            pl.BlockSpec((32, 1), const2),
            pl.BlockSpec((8, 1), const2),
        ],
        out_specs=pl.BlockSpec((G, 1), const2),
        out_shape=jax.ShapeDtypeStruct((G, 1), f32),
        scratch_shapes=[
            pltpu.VMEM((G, H), f32),
            pltpu.VMEM((G, 1), f32),
        ],
    )(x2, batch2d, ones_col, u, fc1w, fc1b, g3, b3, fc2w, fc2b)


def _b8(b):
    return jnp.broadcast_to(b.reshape(1, -1), (8, b.shape[0]))


def kernel(x, edge_index, edge_attr, u, batch, params):
    p = params
    src = edge_index[0]
    dst = edge_index[1]

    wcat1 = jnp.concatenate([p["conv1a_lin_W"], p["conv1b_lin_W"],
                             p["conv1c_lin_W"]], axis=1)
    bcat1 = jnp.concatenate([p["conv1a_lin_b"], p["conv1b_lin_b"],
                             p["conv1c_lin_b"]], axis=0)
    wc1 = p["emlp1_l2_W"] @ wcat1
    bc1 = p["emlp1_l2_b"] @ wcat1 + bcat1
    wcat2 = jnp.concatenate([p["conv2a_lin_W"], p["conv2b_lin_W"],
                             p["conv2c_lin_W"]], axis=1)
    bcat2 = jnp.concatenate([p["conv2a_lin_b"], p["conv2b_lin_b"],
                             p["conv2c_lin_b"]], axis=0)
    wc2 = p["emlp2_l2_W"] @ wcat2
    bc2 = p["emlp2_l2_b"] @ wcat2 + bcat2

    p1c = np.array(list(range(0, 64)) + list(range(128, 192)))
    p2c = np.array(list(range(64, 128)) + list(range(192, 256)))
    p3c = np.array(list(range(256, 384)))
    abc = np.array(list(range(0, 128)))
    cc = np.array(list(range(128, 192)))
    wps = [wc1[:, p1c], wc1[:, p2c], wc1[:, p3c], wc2[:, abc], wc2[:, cc]]
    bps = [_b8(bc1[p1c]), _b8(bc1[p2c]), _b8(bc1[p3c]),
           _b8(bc2[abc]), _b8(bc2[cc])]

    ea_p1, ea_p2, ea_p3, ea_ab, ea_c = _edge_transform(
        edge_attr,
        p["emlp1_l1_W"], _b8(p["emlp1_l1_b"]), wps, bps,
        p["emlp2_l1_W"], _b8(p["emlp2_l1_b"]))

    z128 = jnp.zeros((N, 2 * H), jnp.float32)
    z64 = jnp.zeros((N, H), jnp.float32)
    x_lo = lax.slice(x, (0, 0), (N, H))
    x_hi = lax.slice(x, (0, H), (N, F_IN))

    pp1 = _edge_aggregate([x_lo, x_lo], ea_p1, src, dst, z128, 2)
    pp2 = _edge_aggregate([x_hi, x_hi], ea_p2, src, dst, z128, 2)
    pp3 = _edge_aggregate([x_lo, x_hi], ea_p3, src, dst, z128, 2)
    x1 = _node_update(
        x, [pp1, pp2, pp3],
        [[(0, 0, H), (1, 0, H)],
         [(0, H, H), (1, H, H)],
         [(2, 0, 2 * H)]],
        [p["conv1a_nn1_W"], p["conv1b_nn1_W"], p["conv1c_nn1_W"]],
        [_b8(p["conv1a_nn1_b"]), _b8(p["conv1b_nn1_b"]), _b8(p["conv1c_nn1_b"])],
        [p["conv1a_nn2_W"], p["conv1b_nn2_W"], p["conv1c_nn2_W"]],
        [_b8(p["conv1a_nn2_b"]), _b8(p["conv1b_nn2_b"]), _b8(p["conv1c_nn2_b"])],
        p["lin1_W"], _b8(p["lin1_b"]), _b8(p["ln1_g"]), _b8(p["ln1_beta"]),
        F_IN)

    q_ab = _edge_aggregate([x1, x1], ea_ab, src, dst, z128, 2)
    q_c = _edge_aggregate([x1], ea_c, src, dst, z64, 1)
    x2 = _node_update(
        x1, [q_ab, q_c],
        [[(0, 0, H)], [(0, H, H)], [(1, 0, H)]],
        [p["conv2a_nn1_W"], p["conv2b_nn1_W"], p["conv2c_nn1_W"]],
        [_b8(p["conv2a_nn1_b"]), _b8(p["conv2b_nn1_b"]), _b8(p["conv2c_nn1_b"])],
        [p["conv2a_nn2_W"], p["conv2b_nn2_W"], p["conv2c_nn2_W"]],
        [_b8(p["conv2a_nn2_b"]), _b8(p["conv2b_nn2_b"]), _b8(p["conv2c_nn2_b"])],
        p["lin2_W"], _b8(p["lin2_b"]), _b8(p["ln2_g"]), _b8(p["ln2_beta"]),
        H)

    out = _pool_head(
        x2, batch.reshape(N, 1), jnp.ones((N, 1), jnp.float32), u,
        p["fc1_W"], _b8(p["fc1_b"]), _b8(p["ln3_g"]), _b8(p["ln3_beta"]),
        p["fc2_W"], _b8(p["fc2_b"]))
    return out


# speedup vs baseline: 3.3666x; 1.0055x over previous
import functools

import numpy as np

import jax
import jax.numpy as jnp
from jax import lax
from jax.experimental import pallas as pl
from jax.experimental.pallas import tpu as pltpu
from jax.experimental.pallas import tpu_sc as plsc

N = 10000
E = 320000
F_IN = 128
H = 64
G = 16

NC = 2
NS = 16
NW = NC * NS

ZROW = 624
ZCNT = 640


def _edge_aggregate(tabs, ea, src, dst, zeros, npack):
    W = H * npack
    mesh = plsc.VectorSubcoreMesh(core_axis_name="c", subcore_axis_name="s")
    utabs, slot2tab = [], []
    for t in tabs:
        if not any(t is ut for ut in utabs):
            utabs.append(t)
        slot2tab.append([i for i, ut in enumerate(utabs) if ut is t][0])
    ngather = len(utabs)
    CH = 128 if ngather == 1 else 64
    NB = 2
    NCHT = E // CH
    NCHW = NCHT // NW
    NTAIL = NCHT - NCHW * NW

    @functools.partial(
        pl.kernel,
        mesh=mesh,
        compiler_params=pltpu.CompilerParams(use_tc_tiling_on_sc=False),
        out_type=jax.ShapeDtypeStruct((2 * N, W), jnp.float32),
        scratch_types=[
            [pltpu.VMEM((CH,), jnp.int32)] * NB,
            [pltpu.VMEM((CH,), jnp.int32)] * NB,
            [pltpu.VMEM((CH, H), jnp.float32)] * (NB * ngather),
            [pltpu.VMEM((CH, W), jnp.float32)] * NB,
            pltpu.VMEM_SHARED((N, W), jnp.float32),
            [pltpu.SemaphoreType.DMA] * NB,
            [pltpu.SemaphoreType.DMA] * NB,
            [pltpu.SemaphoreType.DMA] * NB,
            [pltpu.SemaphoreType.DMA] * NB,
        ],
    )
    def k(*refs):
        tab_hbms = refs[:ngather]
        ea_hbm, src_hbm, dst_hbm, z_hbm, out_hbm = refs[ngather:ngather + 5]
        (src_vs, dst_vs, xg_vs, m_vs, aggr_sh,
         sem_meta, sem_ea, sem_g, sem_sc) = refs[ngather + 5:]
        cid = lax.axis_index("c")
        sid = lax.axis_index("s")
        wid = cid * NS + sid
        zbase = sid * ZROW

        pltpu.sync_copy(z_hbm.at[pl.ds(zbase, ZCNT)], aggr_sh.at[pl.ds(zbase, ZCNT)])
        plsc.subcore_barrier()

        def issue_eg(kk, b):
            base = (wid * NCHW + kk) * CH
            c1 = pltpu.async_copy(src_hbm.at[pl.ds(base, CH)], src_vs[b], sem_meta[b])
            c2 = pltpu.async_copy(dst_hbm.at[pl.ds(base, CH)], dst_vs[b], sem_meta[b])
            pltpu.async_copy(ea_hbm.at[pl.ds(base, CH)], m_vs[b], sem_ea[b])
            c1.wait()
            c2.wait()
            for t in range(ngather):
                pltpu.async_copy(tab_hbms[t].at[src_vs[b]], xg_vs[t * NB + b],
                                 sem_g[b])

        def wait_g_ea(b):
            for t in range(ngather):
                pltpu.make_async_copy(tab_hbms[t].at[src_vs[b]],
                                      xg_vs[t * NB + b], sem_g[b]).wait()
            pltpu.make_async_copy(ea_hbm.at[pl.ds(0, CH)], m_vs[b],
                                  sem_ea[b]).wait()

        def compute(b):
            @plsc.parallel_loop(0, CH, unroll=2)
            def _(r):
                for j in range(H // 16):
                    gs = [xg_vs[t * NB + b][r, pl.ds(j * 16, 16)]
                          for t in range(ngather)]
                    for s in range(npack):
                        sl = pl.ds(s * H + j * 16, 16)
                        m_vs[b][r, sl] = jnp.maximum(
                            m_vs[b][r, sl] + gs[slot2tab[s]], 0.0)

        for b in range(NB):
            issue_eg(b, b)

        def pipe_body(m, carry):
            for b in range(NB):
                wait_g_ea(b)
                compute(b)
                pltpu.async_copy(m_vs[b], aggr_sh.at[dst_vs[b]], sem_sc[b],
                                 add=True)
            for b in range(NB):
                pltpu.make_async_copy(m_vs[b], aggr_sh.at[dst_vs[b]],
                                      sem_sc[b]).wait()
                issue_eg(NB * (m + 1) + b, b)
            return carry

        lax.fori_loop(0, NCHW // NB, pipe_body, 0)
        for b in range(NB):
            wait_g_ea(b)

        @pl.when(wid < NTAIL)
        def _():
            base = (NW * NCHW + wid) * CH
            pltpu.sync_copy(src_hbm.at[pl.ds(base, CH)], src_vs[0])
            pltpu.sync_copy(dst_hbm.at[pl.ds(base, CH)], dst_vs[0])
            pltpu.sync_copy(ea_hbm.at[pl.ds(base, CH)], m_vs[0])
            for t in range(ngather):
                pltpu.async_copy(tab_hbms[t].at[src_vs[0]], xg_vs[t * NB],
                                 sem_g[0]).wait()
            compute(0)
            pltpu.sync_copy(m_vs[0], aggr_sh.at[dst_vs[0]], add=True)

        plsc.subcore_barrier()
        pltpu.sync_copy(aggr_sh.at[pl.ds(zbase, ZCNT)],
                        out_hbm.at[pl.ds(cid * N + zbase, ZCNT)])

    return k(*utabs, ea, src, dst, zeros)


def _edge_transform(ea, w1a, b1a, wps, bps, w1b, b1b):
    BE = 3200
    grid = (E // BE,)
    const2 = lambda i: (0, 0)
    row = lambda i: (i, 0)
    widths = [w.shape[1] for w in wps]

    def body(ea_ref, w1a_r, b1a_r, w1b_r, b1b_r, *wbo):
        wrs = wbo[0:5]
        brs = wbo[5:10]
        outs = wbo[10:15]
        e = ea_ref[...]
        t1 = jnp.maximum(
            jnp.dot(e, w1a_r[...], preferred_element_type=jnp.float32)
            + b1a_r[0:1, :], 0.0)
        t2 = jnp.maximum(
            jnp.dot(e, w1b_r[...], preferred_element_type=jnp.float32)
            + b1b_r[0:1, :], 0.0)
        for i, t in enumerate([t1, t1, t1, t2, t2]):
            z = jnp.dot(t, wrs[i][...], preferred_element_type=jnp.float32)
            outs[i][...] = z + brs[i][0:1, :]

    in_specs = [
        pl.BlockSpec((BE, 16), row),
        pl.BlockSpec((16, H), const2),
        pl.BlockSpec((8, H), const2),
        pl.BlockSpec((16, H), const2),
        pl.BlockSpec((8, H), const2),
    ]
    in_specs += [pl.BlockSpec((H, w), const2) for w in widths]
    in_specs += [pl.BlockSpec((8, w), const2) for w in widths]
    return pl.pallas_call(
        body,
        grid=grid,
        in_specs=in_specs,
        out_specs=[pl.BlockSpec((BE, w), row) for w in widths],
        out_shape=[jax.ShapeDtypeStruct((E, w), jnp.float32) for w in widths],
    )(ea, w1a, b1a, w1b, b1b, *wps, *bps)


def _node_update(xin, parts, conv_specs, n1w, n1b, n2w, n2b,
                 l1w, l1b, g, beta, Fin):
    BN = 1000
    grid = (N // BN,)
    row = lambda i: (i, 0)
    shift = lambda i: (i + N // BN, 0)
    const2 = lambda i: (0, 0)
    f32 = jnp.float32
    nparts = len(parts)

    def body(*refs):
        x_r = refs[0]
        prefs = refs[1:1 + 2 * nparts]
        (n1wa, n1wb, n1wc, n1ba, n1bb, n1bc,
         n2wa, n2wb, n2wc, n2ba, n2bb, n2bc,
         l1w_r, l1b_r, g_r, beta_r, out_r) = refs[1 + 2 * nparts:]
        x_b = x_r[...]
        psums = [prefs[2 * ph][...] + prefs[2 * ph + 1][...]
                 for ph in range(nparts)]

        def conv(c, w1, b1, w2, b2):
            segs = [psums[pi][:, off:off + wid] for pi, off, wid in conv_specs[c]]
            h = x_b + (jnp.concatenate(segs, axis=1) if len(segs) > 1 else segs[0])
            t = jnp.maximum(
                jnp.dot(h, w1[...], preferred_element_type=f32) + b1[0:1, :], 0.0)
            o = jnp.dot(t, w2[...], preferred_element_type=f32) + b2[0:1, :]
            return jnp.maximum(o, 0.0)

        cat = jnp.concatenate([
            conv(0, n1wa, n1ba, n2wa, n2ba),
            conv(1, n1wb, n1bb, n2wb, n2bb),
            conv(2, n1wc, n1bc, n2wc, n2bc),
        ], axis=1)
        y = jnp.maximum(
            jnp.dot(cat, l1w_r[...], preferred_element_type=f32) + l1b_r[0:1, :], 0.0)
        m = jnp.mean(y, axis=-1, keepdims=True)
        v = jnp.mean((y - m) ** 2, axis=-1, keepdims=True)
        out_r[...] = (y - m) / jnp.sqrt(v + 1e-5) * g_r[0:1, :] + beta_r[0:1, :]

    in_specs = [pl.BlockSpec((BN, Fin), row)]
    operands = [xin]
    for pt in parts:
        wp = pt.shape[1]
        in_specs += [pl.BlockSpec((BN, wp), row), pl.BlockSpec((BN, wp), shift)]
        operands += [pt, pt]
    in_specs += [pl.BlockSpec((Fin, H), const2)] * 3
    in_specs += [pl.BlockSpec((8, H), const2)] * 3
    in_specs += [pl.BlockSpec((H, H), const2)] * 3
    in_specs += [pl.BlockSpec((8, H), const2)] * 3
    in_specs += [
        pl.BlockSpec((3 * H, H), const2),
        pl.BlockSpec((8, H), const2),
        pl.BlockSpec((8, H), const2),
        pl.BlockSpec((8, H), const2),
    ]
    operands += [n1w[0], n1w[1], n1w[2], n1b[0], n1b[1], n1b[2],
                 n2w[0], n2w[1], n2w[2], n2b[0], n2b[1], n2b[2],
                 l1w, l1b, g, beta]
    return pl.pallas_call(
        body,
        grid=grid,
        in_specs=in_specs,
        out_specs=pl.BlockSpec((BN, H), row),
        out_shape=jax.ShapeDtypeStruct((N, H), f32),
    )(*operands)


def _node_update_pool(xin, parts, conv_specs, n1w, n1b, n2w, n2b,
                      l1w, l1b, g, beta, Fin,
                      batch2d, ones_col, u, fc1w, fc1b, g3, b3, fc2w, fc2b):
    BN = 1000
    grid = (N // BN,)
    row = lambda i: (i, 0)
    shift = lambda i: (i + N // BN, 0)
    const2 = lambda i: (0, 0)
    f32 = jnp.float32
    nparts = len(parts)

    def body(*refs):
        x_r = refs[0]
        prefs = refs[1:1 + 2 * nparts]
        (n1wa, n1wb, n1wc, n1ba, n1bb, n1bc,
         n2wa, n2wb, n2wc, n2ba, n2bb, n2bc,
         l1w_r, l1b_r, g_r, beta_r,
         b_r, one_r, u_r, w1_r, b1_r, g3_r, be3_r, w2_r, b2_r,
         out_r, sums, cnts) = refs[1 + 2 * nparts:]
        x_b = x_r[...]
        psums = [prefs[2 * ph][...] + prefs[2 * ph + 1][...]
                 for ph in range(nparts)]

        def conv(c, w1, b1, w2, b2):
            segs = [psums[pi][:, off:off + wid] for pi, off, wid in conv_specs[c]]
            h = x_b + (jnp.concatenate(segs, axis=1) if len(segs) > 1 else segs[0])
            t = jnp.maximum(
                jnp.dot(h, w1[...], preferred_element_type=f32) + b1[0:1, :], 0.0)
            o = jnp.dot(t, w2[...], preferred_element_type=f32) + b2[0:1, :]
            return jnp.maximum(o, 0.0)

        cat = jnp.concatenate([
            conv(0, n1wa, n1ba, n2wa, n2ba),
            conv(1, n1wb, n1bb, n2wb, n2bb),
            conv(2, n1wc, n1bc, n2wc, n2bc),
        ], axis=1)
        y = jnp.maximum(
            jnp.dot(cat, l1w_r[...], preferred_element_type=f32) + l1b_r[0:1, :], 0.0)
        m = jnp.mean(y, axis=-1, keepdims=True)
        v = jnp.mean((y - m) ** 2, axis=-1, keepdims=True)
        x2_b = (y - m) / jnp.sqrt(v + 1e-5) * g_r[0:1, :] + beta_r[0:1, :]

        i = pl.program_id(0)

        @pl.when(i == 0)
        def _():
            sums[...] = jnp.zeros_like(sums)
            cnts[...] = jnp.zeros_like(cnts)

        oh = (b_r[...] == lax.broadcasted_iota(jnp.int32, (1, G), 1)).astype(f32)
        sums[...] += lax.dot_general(oh, x2_b, (((0,), (0,)), ((), ())),
                                     preferred_element_type=f32)
        cnts[...] += lax.dot_general(oh, one_r[...], (((0,), (0,)), ((), ())),
                                     preferred_element_type=f32)

        @pl.when(i == grid[0] - 1)
        def _():
            mean = sums[...] / jnp.maximum(cnts[...], 1.0)
            xf = jnp.concatenate([mean, u_r[...]], axis=1)
            hh = jnp.maximum(
                jnp.dot(xf, w1_r[...], preferred_element_type=f32) + b1_r[0:1, :],
                0.0)
            mu = jnp.mean(hh, axis=-1, keepdims=True)
            var = jnp.mean((hh - mu) ** 2, axis=-1, keepdims=True)
            hn = (hh - mu) / jnp.sqrt(var + 1e-5) * g3_r[0:1, :] + be3_r[0:1, :]
            out_r[...] = jnp.dot(hn, w2_r[...], preferred_element_type=f32) + b2_r[0:1, :]

    in_specs = [pl.BlockSpec((BN, Fin), row)]
    operands = [xin]
    for pt in parts:
        wp = pt.shape[1]
        in_specs += [pl.BlockSpec((BN, wp), row), pl.BlockSpec((BN, wp), shift)]
        operands += [pt, pt]
    in_specs += [pl.BlockSpec((Fin, H), const2)] * 3
    in_specs += [pl.BlockSpec((8, H), const2)] * 3
    in_specs += [pl.BlockSpec((H, H), const2)] * 3
    in_specs += [pl.BlockSpec((8, H), const2)] * 3
    in_specs += [
        pl.BlockSpec((3 * H, H), const2),
        pl.BlockSpec((8, H), const2),
        pl.BlockSpec((8, H), const2),
        pl.BlockSpec((8, H), const2),
        pl.BlockSpec((BN, 1), row),
        pl.BlockSpec((BN, 1), row),
        pl.BlockSpec((G, 8), const2),
        pl.BlockSpec((H + 8, 32), const2),
        pl.BlockSpec((8, 32), const2),
        pl.BlockSpec((8, 32), const2),
        pl.BlockSpec((8, 32), const2),
        pl.BlockSpec((32, 1), const2),
        pl.BlockSpec((8, 1), const2),
    ]
    operands += [n1w[0], n1w[1], n1w[2], n1b[0], n1b[1], n1b[2],
                 n2w[0], n2w[1], n2w[2], n2b[0], n2b[1], n2b[2],
                 l1w, l1b, g, beta,
                 batch2d, ones_col, u, fc1w, fc1b, g3, b3, fc2w, fc2b]
    return pl.pallas_call(
        body,
        grid=grid,
        in_specs=in_specs,
        out_specs=pl.BlockSpec((G, 1), const2),
        out_shape=jax.ShapeDtypeStruct((G, 1), f32),
        scratch_shapes=[
            pltpu.VMEM((G, H), f32),
            pltpu.VMEM((G, 1), f32),
        ],
    )(*operands)


def _b8(b):
    return jnp.broadcast_to(b.reshape(1, -1), (8, b.shape[0]))


def kernel(x, edge_index, edge_attr, u, batch, params):
    p = params
    src = edge_index[0]
    dst = edge_index[1]

    wcat1 = jnp.concatenate([p["conv1a_lin_W"], p["conv1b_lin_W"],
                             p["conv1c_lin_W"]], axis=1)
    bcat1 = jnp.concatenate([p["conv1a_lin_b"], p["conv1b_lin_b"],
                             p["conv1c_lin_b"]], axis=0)
    wc1 = p["emlp1_l2_W"] @ wcat1
    bc1 = p["emlp1_l2_b"] @ wcat1 + bcat1
    wcat2 = jnp.concatenate([p["conv2a_lin_W"], p["conv2b_lin_W"],
                             p["conv2c_lin_W"]], axis=1)
    bcat2 = jnp.concatenate([p["conv2a_lin_b"], p["conv2b_lin_b"],
                             p["conv2c_lin_b"]], axis=0)
    wc2 = p["emlp2_l2_W"] @ wcat2
    bc2 = p["emlp2_l2_b"] @ wcat2 + bcat2

    p1c = np.array(list(range(0, 64)) + list(range(128, 192)))
    p2c = np.array(list(range(64, 128)) + list(range(192, 256)))
    p3c = np.array(list(range(256, 384)))
    abc = np.array(list(range(0, 128)))
    cc = np.array(list(range(128, 192)))
    wps = [wc1[:, p1c], wc1[:, p2c], wc1[:, p3c], wc2[:, abc], wc2[:, cc]]
    bps = [_b8(bc1[p1c]), _b8(bc1[p2c]), _b8(bc1[p3c]),
           _b8(bc2[abc]), _b8(bc2[cc])]

    ea_p1, ea_p2, ea_p3, ea_ab, ea_c = _edge_transform(
        edge_attr,
        p["emlp1_l1_W"], _b8(p["emlp1_l1_b"]), wps, bps,
        p["emlp2_l1_W"], _b8(p["emlp2_l1_b"]))

    z128 = jnp.zeros((N, 2 * H), jnp.float32)
    z64 = jnp.zeros((N, H), jnp.float32)
    x_lo = lax.slice(x, (0, 0), (N, H))
    x_hi = lax.slice(x, (0, H), (N, F_IN))

    pp1 = _edge_aggregate([x_lo, x_lo], ea_p1, src, dst, z128, 2)
    pp2 = _edge_aggregate([x_hi, x_hi], ea_p2, src, dst, z128, 2)
    pp3 = _edge_aggregate([x_lo, x_hi], ea_p3, src, dst, z128, 2)
    x1 = _node_update(
        x, [pp1, pp2, pp3],
        [[(0, 0, H), (1, 0, H)],
         [(0, H, H), (1, H, H)],
         [(2, 0, 2 * H)]],
        [p["conv1a_nn1_W"], p["conv1b_nn1_W"], p["conv1c_nn1_W"]],
        [_b8(p["conv1a_nn1_b"]), _b8(p["conv1b_nn1_b"]), _b8(p["conv1c_nn1_b"])],
        [p["conv1a_nn2_W"], p["conv1b_nn2_W"], p["conv1c_nn2_W"]],
        [_b8(p["conv1a_nn2_b"]), _b8(p["conv1b_nn2_b"]), _b8(p["conv1c_nn2_b"])],
        p["lin1_W"], _b8(p["lin1_b"]), _b8(p["ln1_g"]), _b8(p["ln1_beta"]),
        F_IN)

    q_ab = _edge_aggregate([x1, x1], ea_ab, src, dst, z128, 2)
    q_c = _edge_aggregate([x1], ea_c, src, dst, z64, 1)
    out = _node_update_pool(
        x1, [q_ab, q_c],
        [[(0, 0, H)], [(0, H, H)], [(1, 0, H)]],
        [p["conv2a_nn1_W"], p["conv2b_nn1_W"], p["conv2c_nn1_W"]],
        [_b8(p["conv2a_nn1_b"]), _b8(p["conv2b_nn1_b"]), _b8(p["conv2c_nn1_b"])],
        [p["conv2a_nn2_W"], p["conv2b_nn2_W"], p["conv2c_nn2_W"]],
        [_b8(p["conv2a_nn2_b"]), _b8(p["conv2b_nn2_b"]), _b8(p["conv2c_nn2_b"])],
        p["lin2_W"], _b8(p["lin2_b"]), _b8(p["ln2_g"]), _b8(p["ln2_beta"]),
        H,
        batch.reshape(N, 1), jnp.ones((N, 1), jnp.float32), u,
        p["fc1_W"], _b8(p["fc1_b"]), _b8(p["ln3_g"]), _b8(p["ln3_beta"]),
        p["fc2_W"], _b8(p["fc2_b"]))
    return out
